# baseline (speedup 1.0000x reference)
import functools

import jax
import jax.numpy as jnp
from jax import lax
from jax.experimental import pallas as pl
from jax.experimental.pallas import tpu as pltpu
from jax.experimental.pallas import tpu_sc as plsc

N = 10000
E = 320000
DIN = 128
DE = 4
H = 10
C = 64
HP = 16
EMB = 10
NL = 3
DOUT = 128
G = 64
F = H * C

NC = 2
NS = 16
NWK = NC * NS
E2 = 327680
EW = E2 // NWK
WIN = 128
NWIN = EW // WIN
NP = 10240
RPT = NP // NS
CH = C // NC
FH = H * CH
EW2 = E2 // NS
NWIN2 = EW2 // WIN

_f32 = jnp.float32

_XPERM = [h * C + p * (C // NC) + i
          for p in range(NC) for h in range(H) for i in range(C // NC)]


def _gelu(x):
    return 0.5 * x * (1.0 + lax.erf(x * (2.0 ** -0.5)))


def _embed_body(x_ref, w_ref, b_ref, o_ref):
    o_ref[...] = x_ref[...] @ w_ref[...] + b_ref[...]


def _tc_embed(x, w, b):
    return pl.pallas_call(
        _embed_body,
        out_shape=jax.ShapeDtypeStruct((N, EMB), _f32),
    )(x, w, b)


def _ea_body(eat_ref, me_ref, aeloop_ref, rn_ref):
    eat = eat_ref[...]
    aeloop_ref[...] = (jnp.sum(eat, axis=1, keepdims=True) / E).T @ me_ref[...]
    rn = jnp.sqrt(jnp.max(jnp.sum(eat * eat, axis=0)))
    rn_ref[...] = jnp.full((1, 1), 0.0, _f32) + rn


def _tc_ea(eat, me_all):
    return pl.pallas_call(
        _ea_body,
        out_shape=(
            jax.ShapeDtypeStruct((1, NL * HP), _f32),
            jax.ShapeDtypeStruct((1, 1), _f32),
        ),
    )(eat, me_all)


_BE = 8192


def _aedge_body(ea_ref, me_ref, o_ref):
    o = ea_ref[...] @ me_ref[...]
    rid = pl.program_id(0) * _BE + lax.broadcasted_iota(jnp.int32, (_BE, HP), 0)
    o_ref[...] = jnp.where(rid < E, o, -120.0)


def _tc_aedge(ea_p, me):
    return pl.pallas_call(
        _aedge_body,
        grid=(E2 // _BE,),
        in_specs=[
            pl.BlockSpec((_BE, DE), lambda i: (i, 0)),
            pl.BlockSpec((DE, HP), lambda i: (0, 0)),
        ],
        out_specs=pl.BlockSpec((_BE, HP), lambda i: (i, 0)),
        out_shape=jax.ShapeDtypeStruct((E2, HP), _f32),
    )(ea_p, me)


def _xs_body(h_ref, w_ref, o_ref):
    o_ref[...] = h_ref[...] @ w_ref[...]


def _tc_xs(h, w):
    return pl.pallas_call(
        _xs_body,
        out_shape=jax.ShapeDtypeStruct((N, F), _f32),
    )(h, w)


def _att_body(h_ref, wsd_ref, rn_ref, menorm_ref, aeloop_ref,
              asd_ref, m_ref, exl_ref):
    h = h_ref[...]
    asd = h @ wsd_ref[...]
    asd_ref[...] = asd
    asrc = asd[:, :HP]
    adst = asd[:, HP:]
    srcmax = jnp.max(asrc, axis=0, keepdims=True)
    dstmax = jnp.max(adst, axis=0, keepdims=True)
    bound = srcmax + dstmax + rn_ref[0, 0] * menorm_ref[...]
    m = jnp.maximum(bound, 0.2 * bound)
    m_ref[...] = m
    al = asrc + adst + aeloop_ref[...]
    al = jnp.maximum(al, 0.2 * al)
    exl = jnp.exp(al - m)
    lane = lax.broadcasted_iota(jnp.int32, (N, HP), 1)
    exl_ref[...] = jnp.where(lane < H, exl, 0.0)


def _tc_att(h, wsd, rn, menorm, aeloop):
    return pl.pallas_call(
        _att_body,
        out_shape=(
            jax.ShapeDtypeStruct((N, 2 * HP), _f32),
            jax.ShapeDtypeStruct((1, HP), _f32),
            jax.ShapeDtypeStruct((N, HP), _f32),
        ),
    )(h, wsd, rn, menorm, aeloop)


def _rden_body(dp_ref, exl_ref, rden_ref, attl_ref):
    exl = exl_ref[...]
    den = dp_ref[0] + dp_ref[1] + exl
    rden = 1.0 / (den + 1e-16)
    rden_ref[...] = rden
    attl_ref[...] = exl * rden


def _tc_rden(dparts, exl):
    return pl.pallas_call(
        _rden_body,
        out_shape=(
            jax.ShapeDtypeStruct((N, HP), _f32),
            jax.ShapeDtypeStruct((N, HP), _f32),
        ),
    )(dparts, exl)


def _outl_body(attl_ref, xs_ref, o_ref):
    attl = attl_ref[...]
    halves = []
    for p in range(NC):
        acc = attl[:, 0:1] * xs_ref[:, p * FH:p * FH + CH]
        for hh in range(1, H):
            acc = acc + attl[:, hh:hh + 1] * xs_ref[:, p * FH + hh * CH:p * FH + (hh + 1) * CH]
        halves.append(acc)
    o_ref[...] = jnp.concatenate(halves, axis=1)


_BN = 1000


def _tc_outl(attl, xs):
    return pl.pallas_call(
        _outl_body,
        grid=(N // _BN,),
        in_specs=[
            pl.BlockSpec((_BN, HP), lambda i: (i, 0)),
            pl.BlockSpec((_BN, F), lambda i: (i, 0)),
        ],
        out_specs=pl.BlockSpec((_BN, C), lambda i: (i, 0)),
        out_shape=jax.ShapeDtypeStruct((N, C), _f32),
    )(attl, xs)


def _post_body(op_ref, ol_ref, bias_ref, g_ref, b_ref, o_ref):
    tot = (jnp.concatenate([op_ref[0], op_ref[1]], axis=1)
           + ol_ref[...]) / H + bias_ref[...]
    mu = jnp.mean(tot, axis=0, keepdims=True)
    var = jnp.mean((tot - mu) * (tot - mu), axis=0, keepdims=True)
    xn = (tot - mu) / jnp.sqrt(var + 1e-5) * g_ref[...] + b_ref[...]
    o_ref[...] = _gelu(xn)


def _tc_post(oparts, outl, bias, bn_g, bn_b):
    return pl.pallas_call(
        _post_body,
        out_shape=jax.ShapeDtypeStruct((N, C), _f32),
    )(oparts, outl, bias, bn_g, bn_b)


def _final_body(h_ref, w1_ref, b1_ref, g_ref, b_ref, w2_ref, b2_ref,
                batch_ref, o_ref):
    y = h_ref[...] @ w1_ref[...] + b1_ref[...]
    mu = jnp.mean(y, axis=0, keepdims=True)
    var = jnp.mean((y - mu) * (y - mu), axis=0, keepdims=True)
    y = (y - mu) / jnp.sqrt(var + 1e-5) * g_ref[...] + b_ref[...]
    y = _gelu(y)
    y = y @ w2_ref[...] + b2_ref[...]
    gid = lax.broadcasted_iota(jnp.int32, (N, G), 1)
    p = jnp.where(batch_ref[...] == gid, 1.0, 0.0)
    s = lax.dot_general(p, y, (((0,), (0,)), ((), ())))
    cnt = lax.dot_general(p, jnp.ones((N, 1), _f32), (((0,), (0,)), ((), ())))
    o_ref[...] = s / jnp.maximum(cnt, 1.0)


def _tc_final(h, w1, b1, g, b, w2, b2, batch_col):
    return pl.pallas_call(
        _final_body,
        out_shape=jax.ShapeDtypeStruct((G, DOUT), _f32),
    )(h, w1, b1, g, b, w2, b2, batch_col)


_MESH = plsc.VectorSubcoreMesh(core_axis_name="c", subcore_axis_name="s")


@functools.partial(
    pl.kernel,
    out_type=(
        jax.ShapeDtypeStruct((E2, HP), _f32),
        jax.ShapeDtypeStruct((NC, NP, HP), _f32),
    ),
    mesh=_MESH,
    compiler_params=pltpu.CompilerParams(use_tc_tiling_on_sc=False),
    scratch_types=[
        pltpu.VMEM_SHARED((NP, HP), _f32),
        pltpu.VMEM((WIN,), jnp.int32),
        pltpu.VMEM((WIN,), jnp.int32),
        pltpu.VMEM((WIN, HP), _f32),
        pltpu.VMEM((1, HP), _f32),
        pltpu.VMEM((WIN, HP), _f32),
        pltpu.VMEM((WIN, HP), _f32),
        pltpu.VMEM((WIN, HP), _f32),
        pltpu.VMEM((RPT, HP), _f32),
    ],
)
def _sc_pass_a(src_e, dst_e, ae, m, asrc_t, adst_t,
               ex_out, dparts,
               den_sp, src_v, dst_v, ae_v, m_v, as_v, ad_v, ex_v, zero_v):
    cid = lax.axis_index("c")
    sid = lax.axis_index("s")
    wid = sid * NC + cid
    base = wid * EW

    zvec = jnp.zeros((HP,), _f32)

    def zb(r, _):
        zero_v[r, :] = zvec
        return 0

    lax.fori_loop(0, RPT, zb, 0)
    pltpu.sync_copy(zero_v, den_sp.at[pl.ds(sid * RPT, RPT)])

    pltpu.sync_copy(m, m_v)
    plsc.subcore_barrier()

    mrow = m_v[0]
    lane = lax.iota(jnp.int32, 16)
    lmask = lane < H

    def wbody(w, _):
        eb = base + w * WIN
        pltpu.sync_copy(src_e.at[pl.ds(eb, WIN)], src_v)
        pltpu.sync_copy(dst_e.at[pl.ds(eb, WIN)], dst_v)
        pltpu.sync_copy(ae.at[pl.ds(eb, WIN)], ae_v)
        pltpu.sync_copy(asrc_t.at[src_v], as_v)
        pltpu.sync_copy(adst_t.at[dst_v], ad_v)

        def ebody(j, _):
            a = as_v[j] + ad_v[j] + ae_v[j]
            a = jnp.maximum(a, 0.2 * a)
            e = jnp.exp(a - mrow)
            ex_v[j, :] = jnp.where(lmask, e, 0.0)
            return 0

        lax.fori_loop(0, WIN, ebody, 0)
        pltpu.sync_copy(ex_v, ex_out.at[pl.ds(eb, WIN)])
        pltpu.sync_copy(ex_v, den_sp.at[dst_v], add=True)
        return 0

    lax.fori_loop(0, NWIN, wbody, 0)

    plsc.subcore_barrier()
    pltpu.sync_copy(den_sp.at[pl.ds(sid * RPT, RPT)],
                    dparts.at[cid, pl.ds(sid * RPT, RPT)])


@functools.partial(
    pl.kernel,
    out_type=jax.ShapeDtypeStruct((NC, NP, CH), _f32),
    mesh=_MESH,
    compiler_params=pltpu.CompilerParams(use_tc_tiling_on_sc=False),
    scratch_types=[
        pltpu.VMEM_SHARED((NP, CH), _f32),
        pltpu.VMEM((WIN,), jnp.int32),
        pltpu.VMEM((WIN,), jnp.int32),
        pltpu.VMEM((WIN, HP), _f32),
        pltpu.VMEM((WIN, HP), _f32),
        pltpu.VMEM((WIN, FH), _f32),
        pltpu.VMEM((WIN, CH), _f32),
        pltpu.VMEM((RPT, CH), _f32),
    ],
)
def _sc_pass_b(idx2, dst_e, ex_t, rden_t, xsr2,
               oparts,
               out_sp, dst_v, idx_v, ex_v, rd_v, xs_v, os_v, zero_v):
    cid = lax.axis_index("c")
    sid = lax.axis_index("s")
    base = sid * EW2

    zvec = jnp.zeros((16,), _f32)

    def zb(r, _):
        for q in range(CH // 16):
            zero_v[r, pl.ds(q * 16, 16)] = zvec
        return 0

    lax.fori_loop(0, RPT, zb, 0)
    pltpu.sync_copy(zero_v, out_sp.at[pl.ds(sid * RPT, RPT)])
    plsc.subcore_barrier()

    def wbody(w, _):
        eb = base + w * WIN
        pltpu.sync_copy(dst_e.at[pl.ds(eb, WIN)], dst_v)
        pltpu.sync_copy(idx2.at[cid, pl.ds(eb, WIN)], idx_v)
        pltpu.sync_copy(ex_t.at[pl.ds(eb, WIN)], ex_v)
        pltpu.sync_copy(rden_t.at[dst_v], rd_v)
        pltpu.sync_copy(xsr2.at[idx_v], xs_v)

        def ebody(j, _):
            att = ex_v[j] * rd_v[j]
            acc0 = jnp.zeros((16,), _f32)
            acc1 = jnp.zeros((16,), _f32)
            for hh in range(H):
                ah = att[hh]
                o = hh * CH
                acc0 = acc0 + ah * xs_v[j, pl.ds(o, 16)]
                acc1 = acc1 + ah * xs_v[j, pl.ds(o + 16, 16)]
            os_v[j, pl.ds(0, 16)] = acc0
            os_v[j, pl.ds(16, 16)] = acc1
            return 0

        lax.fori_loop(0, WIN, ebody, 0)
        pltpu.sync_copy(os_v, out_sp.at[dst_v], add=True)
        return 0

    lax.fori_loop(0, NWIN2, wbody, 0)

    plsc.subcore_barrier()
    pltpu.sync_copy(out_sp.at[pl.ds(sid * RPT, RPT)],
                    oparts.at[cid, pl.ds(sid * RPT, RPT)])


def _pad_cols(a, width):
    return jnp.concatenate(
        [a, jnp.zeros((a.shape[0], width - a.shape[1]), a.dtype)], axis=1)


def kernel(x, edge_index, edge_attr, batch, params):
    me_l = []
    menorm_l = []
    wsd_l = []
    for l in range(NL):
        p = params['convs'][l]
        w_e = p['W_edge'].reshape(DE, H, C)
        me = jnp.einsum('dhc,hc->dh', w_e, p['att_edge'])
        me_l.append(_pad_cols(me, HP))
        menorm_l.append(_pad_cols(
            jnp.sqrt(jnp.sum(me * me, axis=0))[None, :], HP))
        w3 = p['W'].reshape(-1, H, C)
        ws = jnp.einsum('dhc,hc->dh', w3, p['att_src'])
        wd = jnp.einsum('dhc,hc->dh', w3, p['att_dst'])
        wsd_l.append(jnp.concatenate(
            [_pad_cols(ws, HP), _pad_cols(wd, HP)], axis=1))

    me_all = jnp.concatenate(me_l, axis=1)
    eat = edge_attr.T
    aeloop_all, rn = _tc_ea(eat, me_all)

    pad = E2 - E
    zpad = jnp.zeros((pad,), jnp.int32)
    src_e = jnp.concatenate([edge_index[0], zpad])
    dst_e = jnp.concatenate([edge_index[1], zpad])
    idx2 = jnp.stack([src_e * 2, src_e * 2 + 1])
    ea_p = jnp.concatenate([edge_attr, jnp.zeros((pad, DE), _f32)], axis=0)
    h = _tc_embed(x, params['W_emb'], params['b_emb'][None, :])

    for l in range(NL):
        p = params['convs'][l]
        aeloop = aeloop_all[:, l * HP:(l + 1) * HP]
        xs = _tc_xs(h, p['W'][:, _XPERM])
        ae = _tc_aedge(ea_p, me_l[l])
        asd, m, exl = _tc_att(h, wsd_l[l], rn, menorm_l[l], aeloop)
        asrc_t = asd[:, :HP]
        adst_t = asd[:, HP:]
        ex_t, dparts = _sc_pass_a(src_e, dst_e, ae, m, asrc_t, adst_t)
        rden_t, attl = _tc_rden(dparts[:, :N], exl)
        oparts = _sc_pass_b(idx2, dst_e, ex_t, rden_t,
                            xs.reshape(NC * N, FH))
        outl = _tc_outl(attl, xs)
        hc = _tc_post(oparts[:, :N], outl, p['bias'][None, :],
                      p['bn_g'][None, :], p['bn_b'][None, :])
        h = jnp.concatenate([h, hc], axis=1)

    return _tc_final(h, params['W1'], params['b1'][None, :],
                     params['bn_g'][None, :], params['bn_b'][None, :],
                     params['W2'], params['b2'][None, :],
                     batch[:, None].astype(jnp.int32))


# speedup vs baseline: 1.2949x; 1.2949x over previous
import functools

import jax
import jax.numpy as jnp
from jax import lax
from jax.experimental import pallas as pl
from jax.experimental.pallas import tpu as pltpu
from jax.experimental.pallas import tpu_sc as plsc

N = 10000
E = 320000
DIN = 128
DE = 4
H = 10
C = 64
HP = 16
EMB = 10
NL = 3
DOUT = 128
G = 64
F = H * C

NC = 2
NS = 16
NWK = NC * NS
E2 = 327680
EW = E2 // NWK
WIN = 128
NWIN = EW // WIN
NP = 10240
RPT = NP // NS
CH = C // NC
FH = H * CH
EW2 = E2 // NS
NWIN2 = EW2 // WIN

_f32 = jnp.float32

_XPERM = [h * C + p * (C // NC) + i
          for p in range(NC) for h in range(H) for i in range(C // NC)]


def _gelu(x):
    return 0.5 * x * (1.0 + lax.erf(x * (2.0 ** -0.5)))


def _embed_body(x_ref, w_ref, b_ref, o_ref):
    o_ref[...] = x_ref[...] @ w_ref[...] + b_ref[...]


def _tc_embed(x, w, b):
    return pl.pallas_call(
        _embed_body,
        out_shape=jax.ShapeDtypeStruct((N, EMB), _f32),
    )(x, w, b)


def _ea_body(eat_ref, me_ref, aeloop_ref, rn_ref):
    eat = eat_ref[...]
    aeloop_ref[...] = (jnp.sum(eat, axis=1, keepdims=True) / E).T @ me_ref[...]
    rn = jnp.sqrt(jnp.max(jnp.sum(eat * eat, axis=0)))
    rn_ref[...] = jnp.full((1, 1), 0.0, _f32) + rn


def _tc_ea(eat, me_all):
    return pl.pallas_call(
        _ea_body,
        out_shape=(
            jax.ShapeDtypeStruct((1, NL * HP), _f32),
            jax.ShapeDtypeStruct((1, 1), _f32),
        ),
    )(eat, me_all)


_BE = 8192


def _aedge_body(ea_ref, me_ref, o_ref):
    o = ea_ref[...] @ me_ref[...]
    rid = pl.program_id(0) * _BE + lax.broadcasted_iota(jnp.int32, (_BE, HP), 0)
    o_ref[...] = jnp.where(rid < E, o, -120.0)


def _tc_aedge(ea_p, me):
    return pl.pallas_call(
        _aedge_body,
        grid=(E2 // _BE,),
        in_specs=[
            pl.BlockSpec((_BE, DE), lambda i: (i, 0)),
            pl.BlockSpec((DE, HP), lambda i: (0, 0)),
        ],
        out_specs=pl.BlockSpec((_BE, HP), lambda i: (i, 0)),
        out_shape=jax.ShapeDtypeStruct((E2, HP), _f32),
    )(ea_p, me)


def _xs_body(h_ref, w_ref, o_ref):
    o_ref[...] = h_ref[...] @ w_ref[...]


def _tc_xs(h, w):
    return pl.pallas_call(
        _xs_body,
        out_shape=jax.ShapeDtypeStruct((N, F), _f32),
    )(h, w)


def _att_body(h_ref, wsd_ref, rn_ref, menorm_ref, aeloop_ref,
              asd_ref, m_ref, exl_ref):
    h = h_ref[...]
    asd = h @ wsd_ref[...]
    asd_ref[...] = asd
    asrc = asd[:, :HP]
    adst = asd[:, HP:]
    srcmax = jnp.max(asrc, axis=0, keepdims=True)
    dstmax = jnp.max(adst, axis=0, keepdims=True)
    bound = srcmax + dstmax + rn_ref[0, 0] * menorm_ref[...]
    m = jnp.maximum(bound, 0.2 * bound)
    m_ref[...] = m
    al = asrc + adst + aeloop_ref[...]
    al = jnp.maximum(al, 0.2 * al)
    exl = jnp.exp(al - m)
    lane = lax.broadcasted_iota(jnp.int32, (N, HP), 1)
    exl_ref[...] = jnp.where(lane < H, exl, 0.0)


def _tc_att(h, wsd, rn, menorm, aeloop):
    return pl.pallas_call(
        _att_body,
        out_shape=(
            jax.ShapeDtypeStruct((N, 2 * HP), _f32),
            jax.ShapeDtypeStruct((1, HP), _f32),
            jax.ShapeDtypeStruct((N, HP), _f32),
        ),
    )(h, wsd, rn, menorm, aeloop)


def _rden_body(dp_ref, exl_ref, rden_ref, attl_ref):
    exl = exl_ref[...]
    den = dp_ref[0] + dp_ref[1] + exl
    rden = 1.0 / (den + 1e-16)
    rden_ref[...] = rden
    attl_ref[...] = exl * rden


def _tc_rden(dparts, exl):
    return pl.pallas_call(
        _rden_body,
        out_shape=(
            jax.ShapeDtypeStruct((N, HP), _f32),
            jax.ShapeDtypeStruct((N, HP), _f32),
        ),
    )(dparts, exl)


def _outl_body(attl_ref, xs_ref, o_ref):
    attl = attl_ref[...]
    halves = []
    for p in range(NC):
        acc = attl[:, 0:1] * xs_ref[:, p * FH:p * FH + CH]
        for hh in range(1, H):
            acc = acc + attl[:, hh:hh + 1] * xs_ref[:, p * FH + hh * CH:p * FH + (hh + 1) * CH]
        halves.append(acc)
    o_ref[...] = jnp.concatenate(halves, axis=1)


_BN = 1000


def _tc_outl(attl, xs):
    return pl.pallas_call(
        _outl_body,
        grid=(N // _BN,),
        in_specs=[
            pl.BlockSpec((_BN, HP), lambda i: (i, 0)),
            pl.BlockSpec((_BN, F), lambda i: (i, 0)),
        ],
        out_specs=pl.BlockSpec((_BN, C), lambda i: (i, 0)),
        out_shape=jax.ShapeDtypeStruct((N, C), _f32),
    )(attl, xs)


def _post_body(op_ref, ol_ref, bias_ref, g_ref, b_ref, o_ref):
    tot = (jnp.concatenate([op_ref[0], op_ref[1]], axis=1)
           + ol_ref[...]) / H + bias_ref[...]
    mu = jnp.mean(tot, axis=0, keepdims=True)
    var = jnp.mean((tot - mu) * (tot - mu), axis=0, keepdims=True)
    xn = (tot - mu) / jnp.sqrt(var + 1e-5) * g_ref[...] + b_ref[...]
    o_ref[...] = _gelu(xn)


def _tc_post(oparts, outl, bias, bn_g, bn_b):
    return pl.pallas_call(
        _post_body,
        out_shape=jax.ShapeDtypeStruct((N, C), _f32),
    )(oparts, outl, bias, bn_g, bn_b)


def _final_body(h_ref, w1_ref, b1_ref, g_ref, b_ref, w2_ref, b2_ref,
                batch_ref, o_ref):
    y = h_ref[...] @ w1_ref[...] + b1_ref[...]
    mu = jnp.mean(y, axis=0, keepdims=True)
    var = jnp.mean((y - mu) * (y - mu), axis=0, keepdims=True)
    y = (y - mu) / jnp.sqrt(var + 1e-5) * g_ref[...] + b_ref[...]
    y = _gelu(y)
    y = y @ w2_ref[...] + b2_ref[...]
    gid = lax.broadcasted_iota(jnp.int32, (N, G), 1)
    p = jnp.where(batch_ref[...] == gid, 1.0, 0.0)
    s = lax.dot_general(p, y, (((0,), (0,)), ((), ())))
    cnt = lax.dot_general(p, jnp.ones((N, 1), _f32), (((0,), (0,)), ((), ())))
    o_ref[...] = s / jnp.maximum(cnt, 1.0)


def _tc_final(h, w1, b1, g, b, w2, b2, batch_col):
    return pl.pallas_call(
        _final_body,
        out_shape=jax.ShapeDtypeStruct((G, DOUT), _f32),
    )(h, w1, b1, g, b, w2, b2, batch_col)


_MESH = plsc.VectorSubcoreMesh(core_axis_name="c", subcore_axis_name="s")


@functools.partial(
    pl.kernel,
    out_type=(
        jax.ShapeDtypeStruct((E2, HP), _f32),
        jax.ShapeDtypeStruct((NC, NP, HP), _f32),
    ),
    mesh=_MESH,
    compiler_params=pltpu.CompilerParams(use_tc_tiling_on_sc=False),
    scratch_types=[
        pltpu.VMEM_SHARED((NP, HP), _f32),
        pltpu.VMEM((WIN,), jnp.int32),
        pltpu.VMEM((WIN,), jnp.int32),
        pltpu.VMEM((WIN, HP), _f32),
        pltpu.VMEM((1, HP), _f32),
        pltpu.VMEM((WIN, HP), _f32),
        pltpu.VMEM((WIN, HP), _f32),
        pltpu.VMEM((WIN, HP), _f32),
        pltpu.VMEM((RPT, HP), _f32),
    ],
)
def _sc_pass_a(src_e, dst_e, ae, m, asrc_t, adst_t,
               ex_out, dparts,
               den_sp, src_v, dst_v, ae_v, m_v, as_v, ad_v, ex_v, zero_v):
    cid = lax.axis_index("c")
    sid = lax.axis_index("s")
    wid = sid * NC + cid
    base = wid * EW

    zvec = jnp.zeros((HP,), _f32)

    def zb(r, _):
        zero_v[r, :] = zvec
        return 0

    lax.fori_loop(0, RPT, zb, 0)
    pltpu.sync_copy(zero_v, den_sp.at[pl.ds(sid * RPT, RPT)])

    pltpu.sync_copy(m, m_v)
    plsc.subcore_barrier()

    mrow = m_v[0]
    lane = lax.iota(jnp.int32, 16)
    lmask = lane < H

    def wbody(w, _):
        eb = base + w * WIN
        pltpu.sync_copy(src_e.at[pl.ds(eb, WIN)], src_v)
        pltpu.sync_copy(dst_e.at[pl.ds(eb, WIN)], dst_v)
        pltpu.sync_copy(ae.at[pl.ds(eb, WIN)], ae_v)
        pltpu.sync_copy(asrc_t.at[src_v], as_v)
        pltpu.sync_copy(adst_t.at[dst_v], ad_v)

        def ebody(j, _):
            a = as_v[j] + ad_v[j] + ae_v[j]
            a = jnp.maximum(a, 0.2 * a)
            e = jnp.exp(a - mrow)
            ex_v[j, :] = jnp.where(lmask, e, 0.0)
            return 0

        lax.fori_loop(0, WIN, ebody, 0)
        pltpu.sync_copy(ex_v, ex_out.at[pl.ds(eb, WIN)])
        pltpu.sync_copy(ex_v, den_sp.at[dst_v], add=True)
        return 0

    lax.fori_loop(0, NWIN, wbody, 0)

    plsc.subcore_barrier()
    pltpu.sync_copy(den_sp.at[pl.ds(sid * RPT, RPT)],
                    dparts.at[cid, pl.ds(sid * RPT, RPT)])


@functools.partial(
    pl.kernel,
    out_type=jax.ShapeDtypeStruct((NC, NP, CH), _f32),
    mesh=_MESH,
    compiler_params=pltpu.CompilerParams(use_tc_tiling_on_sc=False),
    scratch_types=[
        pltpu.VMEM_SHARED((NP, CH), _f32),
        pltpu.VMEM((WIN,), jnp.int32),
        pltpu.VMEM((WIN,), jnp.int32),
        pltpu.VMEM((WIN, HP), _f32),
        pltpu.VMEM((WIN, HP), _f32),
        pltpu.VMEM((WIN, FH), _f32),
        pltpu.VMEM((WIN, CH), _f32),
        pltpu.VMEM((RPT, CH), _f32),
    ],
)
def _sc_pass_b(idx2, dst_e, ex_t, rden_t, xsr2,
               oparts,
               out_sp, dst_v, idx_v, ex_v, rd_v, xs_v, os_v, zero_v):
    cid = lax.axis_index("c")
    sid = lax.axis_index("s")
    base = sid * EW2

    zvec = jnp.zeros((16,), _f32)

    def zb(r, _):
        for q in range(CH // 16):
            zero_v[r, pl.ds(q * 16, 16)] = zvec
        return 0

    lax.fori_loop(0, RPT, zb, 0)
    pltpu.sync_copy(zero_v, out_sp.at[pl.ds(sid * RPT, RPT)])
    plsc.subcore_barrier()

    def wbody(w, _):
        eb = base + w * WIN
        pltpu.sync_copy(dst_e.at[pl.ds(eb, WIN)], dst_v)
        pltpu.sync_copy(idx2.at[cid, pl.ds(eb, WIN)], idx_v)
        pltpu.sync_copy(ex_t.at[pl.ds(eb, WIN)], ex_v)
        pltpu.sync_copy(rden_t.at[dst_v], rd_v)
        pltpu.sync_copy(xsr2.at[idx_v], xs_v)

        def ebody(j, _):
            att = ex_v[j] * rd_v[j]
            acc0 = jnp.zeros((16,), _f32)
            acc1 = jnp.zeros((16,), _f32)
            for hh in range(H):
                ah = att[hh]
                o = hh * CH
                acc0 = acc0 + ah * xs_v[j, pl.ds(o, 16)]
                acc1 = acc1 + ah * xs_v[j, pl.ds(o + 16, 16)]
            os_v[j, pl.ds(0, 16)] = acc0
            os_v[j, pl.ds(16, 16)] = acc1
            return 0

        lax.fori_loop(0, WIN, ebody, 0)
        pltpu.sync_copy(os_v, out_sp.at[dst_v], add=True)
        return 0

    lax.fori_loop(0, NWIN2, wbody, 0)

    plsc.subcore_barrier()
    pltpu.sync_copy(out_sp.at[pl.ds(sid * RPT, RPT)],
                    oparts.at[cid, pl.ds(sid * RPT, RPT)])


def _pad_cols(a, width):
    return jnp.concatenate(
        [a, jnp.zeros((a.shape[0], width - a.shape[1]), a.dtype)], axis=1)


def kernel(x, edge_index, edge_attr, batch, params):
    me_l = []
    menorm_l = []
    wsd_l = []
    for l in range(NL):
        p = params['convs'][l]
        w_e = p['W_edge'].reshape(DE, H, C)
        me = jnp.einsum('dhc,hc->dh', w_e, p['att_edge'])
        me_l.append(_pad_cols(me, HP))
        menorm_l.append(_pad_cols(
            jnp.sqrt(jnp.sum(me * me, axis=0))[None, :], HP))
        w3 = p['W'].reshape(-1, H, C)
        ws = jnp.einsum('dhc,hc->dh', w3, p['att_src'])
        wd = jnp.einsum('dhc,hc->dh', w3, p['att_dst'])
        wsd_l.append(jnp.concatenate(
            [_pad_cols(ws, HP), _pad_cols(wd, HP)], axis=1))

    me_all = jnp.concatenate(me_l, axis=1)
    eat = edge_attr.T
    aeloop_all, rn = _tc_ea(eat, me_all)

    pad = E2 - E
    spread = jnp.arange(pad, dtype=jnp.int32) * 13 % N
    src_e = jnp.concatenate([edge_index[0], spread])
    dst_e = jnp.concatenate([edge_index[1], spread])
    idx2 = jnp.stack([src_e * 2, src_e * 2 + 1])
    ea_p = jnp.concatenate([edge_attr, jnp.zeros((pad, DE), _f32)], axis=0)
    h = _tc_embed(x, params['W_emb'], params['b_emb'][None, :])

    for l in range(NL):
        p = params['convs'][l]
        aeloop = aeloop_all[:, l * HP:(l + 1) * HP]
        xs = _tc_xs(h, p['W'][:, _XPERM])
        ae = _tc_aedge(ea_p, me_l[l])
        asd, m, exl = _tc_att(h, wsd_l[l], rn, menorm_l[l], aeloop)
        asrc_t = asd[:, :HP]
        adst_t = asd[:, HP:]
        ex_t, dparts = _sc_pass_a(src_e, dst_e, ae, m, asrc_t, adst_t)
        rden_t, attl = _tc_rden(dparts[:, :N], exl)
        oparts = _sc_pass_b(idx2, dst_e, ex_t, rden_t,
                            xs.reshape(NC * N, FH))
        outl = _tc_outl(attl, xs)
        hc = _tc_post(oparts[:, :N], outl, p['bias'][None, :],
                      p['bn_g'][None, :], p['bn_b'][None, :])
        h = jnp.concatenate([h, hc], axis=1)

    return _tc_final(h, params['W1'], params['b1'][None, :],
                     params['bn_g'][None, :], params['bn_b'][None, :],
                     params['W2'], params['b2'][None, :],
                     batch[:, None].astype(jnp.int32))


# speedup vs baseline: 1.6700x; 1.2897x over previous
import functools

import jax
import jax.numpy as jnp
from jax import lax
from jax.experimental import pallas as pl
from jax.experimental.pallas import tpu as pltpu
from jax.experimental.pallas import tpu_sc as plsc

N = 10000
E = 320000
DIN = 128
DE = 4
H = 10
C = 64
HP = 16
EMB = 10
NL = 3
DOUT = 128
G = 64
F = H * C

NC = 2
NS = 16
NWK = NC * NS
E2 = 327680
EW = E2 // NWK
WIN = 128
NWIN = EW // WIN
NP = 10240
RPT = NP // NS
CH = C // NC
FH = H * CH
EW2 = E2 // NS
NWIN2 = EW2 // WIN

_f32 = jnp.float32

_XPERM = [h * C + p * (C // NC) + i
          for p in range(NC) for h in range(H) for i in range(C // NC)]


def _gelu(x):
    return 0.5 * x * (1.0 + lax.erf(x * (2.0 ** -0.5)))


def _embed_body(x_ref, w_ref, b_ref, o_ref):
    o_ref[...] = x_ref[...] @ w_ref[...] + b_ref[...]


def _tc_embed(x, w, b):
    return pl.pallas_call(
        _embed_body,
        out_shape=jax.ShapeDtypeStruct((N, EMB), _f32),
    )(x, w, b)


def _ea_body(eat_ref, me_ref, aeloop_ref, rn_ref):
    eat = eat_ref[...]
    aeloop_ref[...] = (jnp.sum(eat, axis=1, keepdims=True) / E).T @ me_ref[...]
    rn = jnp.sqrt(jnp.max(jnp.sum(eat * eat, axis=0)))
    rn_ref[...] = jnp.full((1, 1), 0.0, _f32) + rn


def _tc_ea(eat, me_all):
    return pl.pallas_call(
        _ea_body,
        out_shape=(
            jax.ShapeDtypeStruct((1, NL * HP), _f32),
            jax.ShapeDtypeStruct((1, 1), _f32),
        ),
    )(eat, me_all)


_BE = 8192


def _aedge_body(ea_ref, me_ref, o_ref):
    o = ea_ref[...] @ me_ref[...]
    rid = pl.program_id(0) * _BE + lax.broadcasted_iota(jnp.int32, (_BE, HP), 0)
    o_ref[...] = jnp.where(rid < E, o, -120.0)


def _tc_aedge(ea_p, me):
    return pl.pallas_call(
        _aedge_body,
        grid=(E2 // _BE,),
        in_specs=[
            pl.BlockSpec((_BE, DE), lambda i: (i, 0)),
            pl.BlockSpec((DE, HP), lambda i: (0, 0)),
        ],
        out_specs=pl.BlockSpec((_BE, HP), lambda i: (i, 0)),
        out_shape=jax.ShapeDtypeStruct((E2, HP), _f32),
    )(ea_p, me)


def _xs_body(h_ref, w_ref, o_ref):
    o_ref[...] = h_ref[...] @ w_ref[...]


def _tc_xs(h, w):
    return pl.pallas_call(
        _xs_body,
        out_shape=jax.ShapeDtypeStruct((N, F), _f32),
    )(h, w)


def _att_body(h_ref, wsd_ref, rn_ref, menorm_ref, aeloop_ref,
              asd_ref, m_ref, exl_ref):
    h = h_ref[...]
    asd = h @ wsd_ref[...]
    asd_ref[...] = asd
    asrc = asd[:, :HP]
    adst = asd[:, HP:]
    srcmax = jnp.max(asrc, axis=0, keepdims=True)
    dstmax = jnp.max(adst, axis=0, keepdims=True)
    bound = srcmax + dstmax + rn_ref[0, 0] * menorm_ref[...]
    m = jnp.maximum(bound, 0.2 * bound)
    m_ref[...] = m
    al = asrc + adst + aeloop_ref[...]
    al = jnp.maximum(al, 0.2 * al)
    exl = jnp.exp(al - m)
    lane = lax.broadcasted_iota(jnp.int32, (N, HP), 1)
    exl_ref[...] = jnp.where(lane < H, exl, 0.0)


def _tc_att(h, wsd, rn, menorm, aeloop):
    return pl.pallas_call(
        _att_body,
        out_shape=(
            jax.ShapeDtypeStruct((N, 2 * HP), _f32),
            jax.ShapeDtypeStruct((1, HP), _f32),
            jax.ShapeDtypeStruct((N, HP), _f32),
        ),
    )(h, wsd, rn, menorm, aeloop)


def _rden_body(dp_ref, exl_ref, rden_ref, attl_ref):
    exl = exl_ref[...]
    den = dp_ref[0] + dp_ref[1] + exl
    rden = 1.0 / (den + 1e-16)
    rden_ref[...] = rden
    attl_ref[...] = exl * rden


def _tc_rden(dparts, exl):
    return pl.pallas_call(
        _rden_body,
        out_shape=(
            jax.ShapeDtypeStruct((N, HP), _f32),
            jax.ShapeDtypeStruct((N, HP), _f32),
        ),
    )(dparts, exl)


def _outl_body(attl_ref, xs_ref, o_ref):
    attl = attl_ref[...]
    halves = []
    for p in range(NC):
        acc = attl[:, 0:1] * xs_ref[:, p * FH:p * FH + CH]
        for hh in range(1, H):
            acc = acc + attl[:, hh:hh + 1] * xs_ref[:, p * FH + hh * CH:p * FH + (hh + 1) * CH]
        halves.append(acc)
    o_ref[...] = jnp.concatenate(halves, axis=1)


_BN = 1000


def _tc_outl(attl, xs):
    return pl.pallas_call(
        _outl_body,
        grid=(N // _BN,),
        in_specs=[
            pl.BlockSpec((_BN, HP), lambda i: (i, 0)),
            pl.BlockSpec((_BN, F), lambda i: (i, 0)),
        ],
        out_specs=pl.BlockSpec((_BN, C), lambda i: (i, 0)),
        out_shape=jax.ShapeDtypeStruct((N, C), _f32),
    )(attl, xs)


def _post_body(op_ref, ol_ref, bias_ref, g_ref, b_ref, o_ref):
    tot = (jnp.concatenate([op_ref[0], op_ref[1]], axis=1)
           + ol_ref[...]) / H + bias_ref[...]
    mu = jnp.mean(tot, axis=0, keepdims=True)
    var = jnp.mean((tot - mu) * (tot - mu), axis=0, keepdims=True)
    xn = (tot - mu) / jnp.sqrt(var + 1e-5) * g_ref[...] + b_ref[...]
    o_ref[...] = _gelu(xn)


def _tc_post(oparts, outl, bias, bn_g, bn_b):
    return pl.pallas_call(
        _post_body,
        out_shape=jax.ShapeDtypeStruct((N, C), _f32),
    )(oparts, outl, bias, bn_g, bn_b)


def _final_body(h_ref, w1_ref, b1_ref, g_ref, b_ref, w2_ref, b2_ref,
                batch_ref, o_ref):
    y = h_ref[...] @ w1_ref[...] + b1_ref[...]
    mu = jnp.mean(y, axis=0, keepdims=True)
    var = jnp.mean((y - mu) * (y - mu), axis=0, keepdims=True)
    y = (y - mu) / jnp.sqrt(var + 1e-5) * g_ref[...] + b_ref[...]
    y = _gelu(y)
    y = y @ w2_ref[...] + b2_ref[...]
    gid = lax.broadcasted_iota(jnp.int32, (N, G), 1)
    p = jnp.where(batch_ref[...] == gid, 1.0, 0.0)
    s = lax.dot_general(p, y, (((0,), (0,)), ((), ())))
    cnt = lax.dot_general(p, jnp.ones((N, 1), _f32), (((0,), (0,)), ((), ())))
    o_ref[...] = s / jnp.maximum(cnt, 1.0)


def _tc_final(h, w1, b1, g, b, w2, b2, batch_col):
    return pl.pallas_call(
        _final_body,
        out_shape=jax.ShapeDtypeStruct((G, DOUT), _f32),
    )(h, w1, b1, g, b, w2, b2, batch_col)


_MESH = plsc.VectorSubcoreMesh(core_axis_name="c", subcore_axis_name="s")


@functools.partial(
    pl.kernel,
    out_type=(
        jax.ShapeDtypeStruct((E2, HP), _f32),
        jax.ShapeDtypeStruct((NC, NP, HP), _f32),
    ),
    mesh=_MESH,
    compiler_params=pltpu.CompilerParams(use_tc_tiling_on_sc=False),
    scratch_types=[
        pltpu.VMEM_SHARED((NP, HP), _f32),
        pltpu.VMEM((WIN,), jnp.int32),
        pltpu.VMEM((WIN,), jnp.int32),
        pltpu.VMEM((WIN, HP), _f32),
        pltpu.VMEM((1, HP), _f32),
        pltpu.VMEM((WIN, HP), _f32),
        pltpu.VMEM((WIN, HP), _f32),
        pltpu.VMEM((WIN, HP), _f32),
        pltpu.VMEM((RPT, HP), _f32),
        pltpu.SemaphoreType.DMA,
        pltpu.SemaphoreType.DMA,
        pltpu.SemaphoreType.DMA,
    ],
)
def _sc_pass_a(src_e, dst_e, ae, m, asrc_t, adst_t,
               ex_out, dparts,
               den_sp, src_v, dst_v, ae_v, m_v, as_v, ad_v, ex_v, zero_v,
               sem1, sem2, sem3):
    cid = lax.axis_index("c")
    sid = lax.axis_index("s")
    wid = sid * NC + cid
    base = wid * EW

    zvec = jnp.zeros((HP,), _f32)

    def zb(r, _):
        zero_v[r, :] = zvec
        return 0

    lax.fori_loop(0, RPT, zb, 0)
    pltpu.sync_copy(zero_v, den_sp.at[pl.ds(sid * RPT, RPT)])

    pltpu.sync_copy(m, m_v)
    plsc.subcore_barrier()

    mrow = m_v[0]
    lane = lax.iota(jnp.int32, 16)
    lmask = lane < H

    def wbody(w, _):
        eb = base + w * WIN
        c1 = pltpu.async_copy(src_e.at[pl.ds(eb, WIN)], src_v, sem1)
        c2 = pltpu.async_copy(dst_e.at[pl.ds(eb, WIN)], dst_v, sem2)
        c3 = pltpu.async_copy(ae.at[pl.ds(eb, WIN)], ae_v, sem3)
        c1.wait()
        c2.wait()
        c3.wait()
        g1 = pltpu.async_copy(asrc_t.at[src_v], as_v, sem1)
        g2 = pltpu.async_copy(adst_t.at[dst_v], ad_v, sem2)
        g1.wait()
        g2.wait()

        def ebody(j, _):
            a = as_v[j] + ad_v[j] + ae_v[j]
            a = jnp.maximum(a, 0.2 * a)
            e = jnp.exp(a - mrow)
            ex_v[j, :] = jnp.where(lmask, e, 0.0)
            return 0

        lax.fori_loop(0, WIN, ebody, 0)
        w1 = pltpu.async_copy(ex_v, ex_out.at[pl.ds(eb, WIN)], sem1)
        w2 = pltpu.async_copy(ex_v, den_sp.at[dst_v], sem2, add=True)
        w1.wait()
        w2.wait()
        return 0

    lax.fori_loop(0, NWIN, wbody, 0)

    plsc.subcore_barrier()
    pltpu.sync_copy(den_sp.at[pl.ds(sid * RPT, RPT)],
                    dparts.at[cid, pl.ds(sid * RPT, RPT)])


@functools.partial(
    pl.kernel,
    out_type=jax.ShapeDtypeStruct((NC, NP, CH), _f32),
    mesh=_MESH,
    compiler_params=pltpu.CompilerParams(use_tc_tiling_on_sc=False),
    scratch_types=[
        pltpu.VMEM_SHARED((NP, CH), _f32),
        pltpu.VMEM((WIN,), jnp.int32),
        pltpu.VMEM((WIN,), jnp.int32),
        pltpu.VMEM((WIN, HP), _f32),
        pltpu.VMEM((WIN, HP), _f32),
        pltpu.VMEM((WIN, FH), _f32),
        pltpu.VMEM((WIN, CH), _f32),
        pltpu.VMEM((RPT, CH), _f32),
        pltpu.SemaphoreType.DMA,
        pltpu.SemaphoreType.DMA,
        pltpu.SemaphoreType.DMA,
    ],
)
def _sc_pass_b(idx2, dst_e, ex_t, rden_t, xsr2,
               oparts,
               out_sp, dst_v, idx_v, ex_v, rd_v, xs_v, os_v, zero_v,
               sem1, sem2, sem3):
    cid = lax.axis_index("c")
    sid = lax.axis_index("s")
    base = sid * EW2

    zvec = jnp.zeros((16,), _f32)

    def zb(r, _):
        for q in range(CH // 16):
            zero_v[r, pl.ds(q * 16, 16)] = zvec
        return 0

    lax.fori_loop(0, RPT, zb, 0)
    pltpu.sync_copy(zero_v, out_sp.at[pl.ds(sid * RPT, RPT)])
    plsc.subcore_barrier()

    def wbody(w, _):
        eb = base + w * WIN
        c1 = pltpu.async_copy(dst_e.at[pl.ds(eb, WIN)], dst_v, sem1)
        c2 = pltpu.async_copy(idx2.at[cid, pl.ds(eb, WIN)], idx_v, sem2)
        c3 = pltpu.async_copy(ex_t.at[pl.ds(eb, WIN)], ex_v, sem3)
        c1.wait()
        c2.wait()
        c3.wait()
        g1 = pltpu.async_copy(rden_t.at[dst_v], rd_v, sem1)
        g2 = pltpu.async_copy(xsr2.at[idx_v], xs_v, sem2)
        g1.wait()
        g2.wait()

        def ebody(j, _):
            att = ex_v[j] * rd_v[j]
            acc0 = jnp.zeros((16,), _f32)
            acc1 = jnp.zeros((16,), _f32)
            for hh in range(H):
                ah = att[hh]
                o = hh * CH
                acc0 = acc0 + ah * xs_v[j, pl.ds(o, 16)]
                acc1 = acc1 + ah * xs_v[j, pl.ds(o + 16, 16)]
            os_v[j, pl.ds(0, 16)] = acc0
            os_v[j, pl.ds(16, 16)] = acc1
            return 0

        lax.fori_loop(0, WIN, ebody, 0)
        pltpu.sync_copy(os_v, out_sp.at[dst_v], add=True)
        return 0

    lax.fori_loop(0, NWIN2, wbody, 0)

    plsc.subcore_barrier()
    pltpu.sync_copy(out_sp.at[pl.ds(sid * RPT, RPT)],
                    oparts.at[cid, pl.ds(sid * RPT, RPT)])


def _pad_cols(a, width):
    return jnp.concatenate(
        [a, jnp.zeros((a.shape[0], width - a.shape[1]), a.dtype)], axis=1)


def kernel(x, edge_index, edge_attr, batch, params):
    me_l = []
    menorm_l = []
    wsd_l = []
    for l in range(NL):
        p = params['convs'][l]
        w_e = p['W_edge'].reshape(DE, H, C)
        me = jnp.einsum('dhc,hc->dh', w_e, p['att_edge'])
        me_l.append(_pad_cols(me, HP))
        menorm_l.append(_pad_cols(
            jnp.sqrt(jnp.sum(me * me, axis=0))[None, :], HP))
        w3 = p['W'].reshape(-1, H, C)
        ws = jnp.einsum('dhc,hc->dh', w3, p['att_src'])
        wd = jnp.einsum('dhc,hc->dh', w3, p['att_dst'])
        wsd_l.append(jnp.concatenate(
            [_pad_cols(ws, HP), _pad_cols(wd, HP)], axis=1))

    me_all = jnp.concatenate(me_l, axis=1)
    eat = edge_attr.T
    aeloop_all, rn = _tc_ea(eat, me_all)

    pad = E2 - E
    spread = jnp.arange(pad, dtype=jnp.int32) * 13 % N
    src_e = jnp.concatenate([edge_index[0], spread])
    dst_e = jnp.concatenate([edge_index[1], spread])
    idx2 = jnp.stack([src_e * 2, src_e * 2 + 1])
    ea_p = jnp.concatenate([edge_attr, jnp.zeros((pad, DE), _f32)], axis=0)
    h = _tc_embed(x, params['W_emb'], params['b_emb'][None, :])

    for l in range(NL):
        p = params['convs'][l]
        aeloop = aeloop_all[:, l * HP:(l + 1) * HP]
        xs = _tc_xs(h, p['W'][:, _XPERM])
        ae = _tc_aedge(ea_p, me_l[l])
        asd, m, exl = _tc_att(h, wsd_l[l], rn, menorm_l[l], aeloop)
        asrc_t = asd[:, :HP]
        adst_t = asd[:, HP:]
        ex_t, dparts = _sc_pass_a(src_e, dst_e, ae, m, asrc_t, adst_t)
        rden_t, attl = _tc_rden(dparts[:, :N], exl)
        oparts = _sc_pass_b(idx2, dst_e, ex_t, rden_t,
                            xs.reshape(NC * N, FH))
        outl = _tc_outl(attl, xs)
        hc = _tc_post(oparts[:, :N], outl, p['bias'][None, :],
                      p['bn_g'][None, :], p['bn_b'][None, :])
        h = jnp.concatenate([h, hc], axis=1)

    return _tc_final(h, params['W1'], params['b1'][None, :],
                     params['bn_g'][None, :], params['bn_b'][None, :],
                     params['W2'], params['b2'][None, :],
                     batch[:, None].astype(jnp.int32))


# speedup vs baseline: 2.2095x; 1.3231x over previous
import functools

import jax
import jax.numpy as jnp
from jax import lax
from jax.experimental import pallas as pl
from jax.experimental.pallas import tpu as pltpu
from jax.experimental.pallas import tpu_sc as plsc

N = 10000
E = 320000
DIN = 128
DE = 4
H = 10
C = 64
HP = 16
EMB = 10
NL = 3
DOUT = 128
G = 64
F = H * C

NC = 2
NS = 16
NWK = NC * NS
E2 = 327680
EW = E2 // NWK
WIN = 128
NWIN = EW // WIN
NP = 10240
RPT = NP // NS
CH = C // NC
FH = H * CH
EW2 = E2 // NS
NWIN2 = EW2 // WIN

_f32 = jnp.float32

_XPERM = [h * C + p * (C // NC) + i
          for p in range(NC) for h in range(H) for i in range(C // NC)]


def _gelu(x):
    return 0.5 * x * (1.0 + lax.erf(x * (2.0 ** -0.5)))


def _embed_body(x_ref, w_ref, b_ref, o_ref):
    o_ref[...] = x_ref[...] @ w_ref[...] + b_ref[...]


def _tc_embed(x, w, b):
    return pl.pallas_call(
        _embed_body,
        out_shape=jax.ShapeDtypeStruct((N, EMB), _f32),
    )(x, w, b)


def _ea_body(eat_ref, me_ref, aeloop_ref, rn_ref):
    eat = eat_ref[...]
    aeloop_ref[...] = (jnp.sum(eat, axis=1, keepdims=True) / E).T @ me_ref[...]
    rn = jnp.sqrt(jnp.max(jnp.sum(eat * eat, axis=0)))
    rn_ref[...] = jnp.full((1, 1), 0.0, _f32) + rn


def _tc_ea(eat, me_all):
    return pl.pallas_call(
        _ea_body,
        out_shape=(
            jax.ShapeDtypeStruct((1, NL * HP), _f32),
            jax.ShapeDtypeStruct((1, 1), _f32),
        ),
    )(eat, me_all)


_BE = 8192


def _aedge_body(ea_ref, me_ref, o_ref):
    o = ea_ref[...] @ me_ref[...]
    rid = pl.program_id(0) * _BE + lax.broadcasted_iota(jnp.int32, (_BE, HP), 0)
    o_ref[...] = jnp.where(rid < E, o, -120.0)


def _tc_aedge(ea_p, me):
    return pl.pallas_call(
        _aedge_body,
        grid=(E2 // _BE,),
        in_specs=[
            pl.BlockSpec((_BE, DE), lambda i: (i, 0)),
            pl.BlockSpec((DE, HP), lambda i: (0, 0)),
        ],
        out_specs=pl.BlockSpec((_BE, HP), lambda i: (i, 0)),
        out_shape=jax.ShapeDtypeStruct((E2, HP), _f32),
    )(ea_p, me)


def _xs_body(h_ref, w_ref, o_ref):
    o_ref[...] = h_ref[...] @ w_ref[...]


def _tc_xs(h, w):
    return pl.pallas_call(
        _xs_body,
        out_shape=jax.ShapeDtypeStruct((N, F), _f32),
    )(h, w)


def _att_body(h_ref, wsd_ref, rn_ref, menorm_ref, aeloop_ref,
              asd_ref, m_ref, exl_ref):
    h = h_ref[...]
    asd = h @ wsd_ref[...]
    asd_ref[...] = asd
    asrc = asd[:, :HP]
    adst = asd[:, HP:]
    srcmax = jnp.max(asrc, axis=0, keepdims=True)
    dstmax = jnp.max(adst, axis=0, keepdims=True)
    bound = srcmax + dstmax + rn_ref[0, 0] * menorm_ref[...]
    m = jnp.maximum(bound, 0.2 * bound)
    m_ref[...] = m
    al = asrc + adst + aeloop_ref[...]
    al = jnp.maximum(al, 0.2 * al)
    exl = jnp.exp(al - m)
    lane = lax.broadcasted_iota(jnp.int32, (N, HP), 1)
    exl_ref[...] = jnp.where(lane < H, exl, 0.0)


def _tc_att(h, wsd, rn, menorm, aeloop):
    return pl.pallas_call(
        _att_body,
        out_shape=(
            jax.ShapeDtypeStruct((N, 2 * HP), _f32),
            jax.ShapeDtypeStruct((1, HP), _f32),
            jax.ShapeDtypeStruct((N, HP), _f32),
        ),
    )(h, wsd, rn, menorm, aeloop)


def _rden_body(dp_ref, exl_ref, rden_ref, attl_ref):
    exl = exl_ref[...]
    den = dp_ref[0] + dp_ref[1] + exl
    rden = 1.0 / (den + 1e-16)
    rden_ref[...] = rden
    attl_ref[...] = exl * rden


def _tc_rden(dparts, exl):
    return pl.pallas_call(
        _rden_body,
        out_shape=(
            jax.ShapeDtypeStruct((N, HP), _f32),
            jax.ShapeDtypeStruct((N, HP), _f32),
        ),
    )(dparts, exl)


def _outl_body(attl_ref, xs_ref, o_ref):
    attl = attl_ref[...]
    halves = []
    for p in range(NC):
        acc = attl[:, 0:1] * xs_ref[:, p * FH:p * FH + CH]
        for hh in range(1, H):
            acc = acc + attl[:, hh:hh + 1] * xs_ref[:, p * FH + hh * CH:p * FH + (hh + 1) * CH]
        halves.append(acc)
    o_ref[...] = jnp.concatenate(halves, axis=1)


_BN = 1000


def _tc_outl(attl, xs):
    return pl.pallas_call(
        _outl_body,
        grid=(N // _BN,),
        in_specs=[
            pl.BlockSpec((_BN, HP), lambda i: (i, 0)),
            pl.BlockSpec((_BN, F), lambda i: (i, 0)),
        ],
        out_specs=pl.BlockSpec((_BN, C), lambda i: (i, 0)),
        out_shape=jax.ShapeDtypeStruct((N, C), _f32),
    )(attl, xs)


def _post_body(op_ref, ol_ref, bias_ref, g_ref, b_ref, o_ref):
    tot = (jnp.concatenate([op_ref[0], op_ref[1]], axis=1)
           + ol_ref[...]) / H + bias_ref[...]
    mu = jnp.mean(tot, axis=0, keepdims=True)
    var = jnp.mean((tot - mu) * (tot - mu), axis=0, keepdims=True)
    xn = (tot - mu) / jnp.sqrt(var + 1e-5) * g_ref[...] + b_ref[...]
    o_ref[...] = _gelu(xn)


def _tc_post(oparts, outl, bias, bn_g, bn_b):
    return pl.pallas_call(
        _post_body,
        out_shape=jax.ShapeDtypeStruct((N, C), _f32),
    )(oparts, outl, bias, bn_g, bn_b)


def _final_body(h_ref, w1_ref, b1_ref, g_ref, b_ref, w2_ref, b2_ref,
                batch_ref, o_ref):
    y = h_ref[...] @ w1_ref[...] + b1_ref[...]
    mu = jnp.mean(y, axis=0, keepdims=True)
    var = jnp.mean((y - mu) * (y - mu), axis=0, keepdims=True)
    y = (y - mu) / jnp.sqrt(var + 1e-5) * g_ref[...] + b_ref[...]
    y = _gelu(y)
    y = y @ w2_ref[...] + b2_ref[...]
    gid = lax.broadcasted_iota(jnp.int32, (N, G), 1)
    p = jnp.where(batch_ref[...] == gid, 1.0, 0.0)
    s = lax.dot_general(p, y, (((0,), (0,)), ((), ())))
    cnt = lax.dot_general(p, jnp.ones((N, 1), _f32), (((0,), (0,)), ((), ())))
    o_ref[...] = s / jnp.maximum(cnt, 1.0)


def _tc_final(h, w1, b1, g, b, w2, b2, batch_col):
    return pl.pallas_call(
        _final_body,
        out_shape=jax.ShapeDtypeStruct((G, DOUT), _f32),
    )(h, w1, b1, g, b, w2, b2, batch_col)


_MESH = plsc.VectorSubcoreMesh(core_axis_name="c", subcore_axis_name="s")


@functools.partial(
    pl.kernel,
    out_type=(
        jax.ShapeDtypeStruct((E2, HP), _f32),
        jax.ShapeDtypeStruct((NC, NP, HP), _f32),
    ),
    mesh=_MESH,
    compiler_params=pltpu.CompilerParams(use_tc_tiling_on_sc=False),
    scratch_types=[
        pltpu.VMEM_SHARED((NP, HP), _f32),
        pltpu.VMEM((WIN,), jnp.int32),
        pltpu.VMEM((WIN,), jnp.int32),
        pltpu.VMEM((WIN, HP), _f32),
        pltpu.VMEM((1, HP), _f32),
        pltpu.VMEM((WIN, HP), _f32),
        pltpu.VMEM((WIN, HP), _f32),
        pltpu.VMEM((WIN, HP), _f32),
        pltpu.VMEM((RPT, HP), _f32),
        pltpu.SemaphoreType.DMA,
        pltpu.SemaphoreType.DMA,
        pltpu.SemaphoreType.DMA,
    ],
)
def _sc_pass_a(src_e, dst_e, ae, m, asrc_t, adst_t,
               ex_out, dparts,
               den_sp, src_v, dst_v, ae_v, m_v, as_v, ad_v, ex_v, zero_v,
               sem1, sem2, sem3):
    cid = lax.axis_index("c")
    sid = lax.axis_index("s")
    wid = sid * NC + cid
    base = wid * EW

    zvec = jnp.zeros((HP,), _f32)

    def zb(r, _):
        zero_v[r, :] = zvec
        return 0

    lax.fori_loop(0, RPT, zb, 0)
    pltpu.sync_copy(zero_v, den_sp.at[pl.ds(sid * RPT, RPT)])

    pltpu.sync_copy(m, m_v)
    plsc.subcore_barrier()

    mrow = m_v[0]
    lane = lax.iota(jnp.int32, 16)
    lmask = lane < H

    def wbody(w, _):
        eb = base + w * WIN
        c1 = pltpu.async_copy(src_e.at[pl.ds(eb, WIN)], src_v, sem1)
        c2 = pltpu.async_copy(dst_e.at[pl.ds(eb, WIN)], dst_v, sem2)
        c3 = pltpu.async_copy(ae.at[pl.ds(eb, WIN)], ae_v, sem3)
        c1.wait()
        c2.wait()
        c3.wait()
        g1 = pltpu.async_copy(asrc_t.at[src_v], as_v, sem1)
        g2 = pltpu.async_copy(adst_t.at[dst_v], ad_v, sem2)
        g1.wait()
        g2.wait()

        def ebody(j, _):
            a = as_v[j] + ad_v[j] + ae_v[j]
            a = jnp.maximum(a, 0.2 * a)
            e = jnp.exp(a - mrow)
            ex_v[j, :] = jnp.where(lmask, e, 0.0)
            return 0

        lax.fori_loop(0, WIN, ebody, 0)
        w1 = pltpu.async_copy(ex_v, ex_out.at[pl.ds(eb, WIN)], sem1)
        w2 = pltpu.async_copy(ex_v, den_sp.at[dst_v], sem2, add=True)
        w1.wait()
        w2.wait()
        return 0

    lax.fori_loop(0, NWIN, wbody, 0)

    plsc.subcore_barrier()
    pltpu.sync_copy(den_sp.at[pl.ds(sid * RPT, RPT)],
                    dparts.at[cid, pl.ds(sid * RPT, RPT)])


@functools.partial(
    pl.kernel,
    out_type=jax.ShapeDtypeStruct((NC, NP, CH), _f32),
    mesh=_MESH,
    compiler_params=pltpu.CompilerParams(use_tc_tiling_on_sc=False),
    scratch_types=[
        pltpu.VMEM_SHARED((NP, CH), _f32),
        pltpu.VMEM((WIN,), jnp.int32),
        pltpu.VMEM((WIN,), jnp.int32),
        pltpu.VMEM((WIN,), jnp.int32),
        pltpu.VMEM((WIN,), jnp.int32),
        pltpu.VMEM((WIN, HP), _f32),
        pltpu.VMEM((WIN, HP), _f32),
        pltpu.VMEM((WIN, HP), _f32),
        pltpu.VMEM((WIN, HP), _f32),
        pltpu.VMEM((WIN, FH), _f32),
        pltpu.VMEM((WIN, FH), _f32),
        pltpu.VMEM((WIN, CH), _f32),
        pltpu.VMEM((WIN, CH), _f32),
        pltpu.SemaphoreType.DMA,
        pltpu.SemaphoreType.DMA,
        pltpu.SemaphoreType.DMA,
        pltpu.SemaphoreType.DMA,
        pltpu.SemaphoreType.DMA,
        pltpu.SemaphoreType.DMA,
        pltpu.SemaphoreType.DMA,
        pltpu.SemaphoreType.DMA,
        pltpu.SemaphoreType.DMA,
        pltpu.SemaphoreType.DMA,
    ],
)
def _sc_pass_b(idx2, dst_e, ex_t, rden_t, xsr2,
               oparts,
               out_sp, dst_v0, dst_v1, idx_v0, idx_v1, ex_v0, ex_v1,
               rd_v0, rd_v1, xs_v0, xs_v1, os_v, zero_c,
               slA0, slB0, slC0, slA1, slB1, slC1, sgA0, sgB0, sgA1, sgB1):
    cid = lax.axis_index("c")
    sid = lax.axis_index("s")
    base = sid * EW2

    dst_v = (dst_v0, dst_v1)
    idx_v = (idx_v0, idx_v1)
    ex_v = (ex_v0, ex_v1)
    rd_v = (rd_v0, rd_v1)
    xs_v = (xs_v0, xs_v1)
    slA = (slA0, slA1)
    slB = (slB0, slB1)
    slC = (slC0, slC1)
    sgA = (sgA0, sgA1)
    sgB = (sgB0, sgB1)

    zvec = jnp.zeros((16,), _f32)

    def zb(r, _):
        for q in range(CH // 16):
            zero_c[r, pl.ds(q * 16, 16)] = zvec
        return 0

    lax.fori_loop(0, WIN, zb, 0)
    for k in range(RPT // WIN):
        pltpu.sync_copy(zero_c, out_sp.at[pl.ds(sid * RPT + k * WIN, WIN)])
    plsc.subcore_barrier()

    def issue_loads(w, p):
        eb = base + w * WIN
        pltpu.async_copy(dst_e.at[pl.ds(eb, WIN)], dst_v[p], slA[p])
        pltpu.async_copy(idx2.at[cid, pl.ds(eb, WIN)], idx_v[p], slB[p])
        pltpu.async_copy(ex_t.at[pl.ds(eb, WIN)], ex_v[p], slC[p])

    def wait_loads(p):
        pltpu.make_async_copy(dst_e.at[pl.ds(0, WIN)], dst_v[p], slA[p]).wait()
        pltpu.make_async_copy(idx2.at[0, pl.ds(0, WIN)], idx_v[p], slB[p]).wait()
        pltpu.make_async_copy(ex_t.at[pl.ds(0, WIN)], ex_v[p], slC[p]).wait()

    def issue_gathers(p):
        pltpu.async_copy(rden_t.at[dst_v[p]], rd_v[p], sgA[p])
        pltpu.async_copy(xsr2.at[idx_v[p]], xs_v[p], sgB[p])

    def wait_gathers(p):
        pltpu.make_async_copy(rden_t.at[dst_v[p]], rd_v[p], sgA[p]).wait()
        pltpu.make_async_copy(xsr2.at[idx_v[p]], xs_v[p], sgB[p]).wait()

    issue_loads(0, 0)
    issue_loads(1, 1)
    wait_loads(0)
    issue_gathers(0)

    def wbody(w2, _):
        for p in range(2):
            w = w2 * 2 + p
            q = 1 - p
            wait_gathers(p)

            @pl.when(w + 1 < NWIN2)
            def _():
                wait_loads(q)
                issue_gathers(q)

            def ebody(j, _):
                att = ex_v[p][j] * rd_v[p][j]
                acc0 = jnp.zeros((16,), _f32)
                acc1 = jnp.zeros((16,), _f32)
                for hh in range(H):
                    ah = att[hh]
                    o = hh * CH
                    acc0 = acc0 + ah * xs_v[p][j, pl.ds(o, 16)]
                    acc1 = acc1 + ah * xs_v[p][j, pl.ds(o + 16, 16)]
                os_v[j, pl.ds(0, 16)] = acc0
                os_v[j, pl.ds(16, 16)] = acc1
                return 0

            lax.fori_loop(0, WIN, ebody, 0)
            pltpu.sync_copy(os_v, out_sp.at[dst_v[p]], add=True)

            @pl.when(w + 2 < NWIN2)
            def _():
                issue_loads(w + 2, p)
        return 0

    lax.fori_loop(0, NWIN2 // 2, wbody, 0)

    plsc.subcore_barrier()
    pltpu.sync_copy(out_sp.at[pl.ds(sid * RPT, RPT)],
                    oparts.at[cid, pl.ds(sid * RPT, RPT)])


def _pad_cols(a, width):
    return jnp.concatenate(
        [a, jnp.zeros((a.shape[0], width - a.shape[1]), a.dtype)], axis=1)


def kernel(x, edge_index, edge_attr, batch, params):
    me_l = []
    menorm_l = []
    wsd_l = []
    for l in range(NL):
        p = params['convs'][l]
        w_e = p['W_edge'].reshape(DE, H, C)
        me = jnp.einsum('dhc,hc->dh', w_e, p['att_edge'])
        me_l.append(_pad_cols(me, HP))
        menorm_l.append(_pad_cols(
            jnp.sqrt(jnp.sum(me * me, axis=0))[None, :], HP))
        w3 = p['W'].reshape(-1, H, C)
        ws = jnp.einsum('dhc,hc->dh', w3, p['att_src'])
        wd = jnp.einsum('dhc,hc->dh', w3, p['att_dst'])
        wsd_l.append(jnp.concatenate(
            [_pad_cols(ws, HP), _pad_cols(wd, HP)], axis=1))

    me_all = jnp.concatenate(me_l, axis=1)
    eat = edge_attr.T
    aeloop_all, rn = _tc_ea(eat, me_all)

    pad = E2 - E
    spread = jnp.arange(pad, dtype=jnp.int32) * 13 % N
    src_e = jnp.concatenate([edge_index[0], spread])
    dst_e = jnp.concatenate([edge_index[1], spread])
    idx2 = jnp.stack([src_e * 2, src_e * 2 + 1])
    ea_p = jnp.concatenate([edge_attr, jnp.zeros((pad, DE), _f32)], axis=0)
    h = _tc_embed(x, params['W_emb'], params['b_emb'][None, :])

    for l in range(NL):
        p = params['convs'][l]
        aeloop = aeloop_all[:, l * HP:(l + 1) * HP]
        xs = _tc_xs(h, p['W'][:, _XPERM])
        ae = _tc_aedge(ea_p, me_l[l])
        asd, m, exl = _tc_att(h, wsd_l[l], rn, menorm_l[l], aeloop)
        asrc_t = asd[:, :HP]
        adst_t = asd[:, HP:]
        ex_t, dparts = _sc_pass_a(src_e, dst_e, ae, m, asrc_t, adst_t)
        rden_t, attl = _tc_rden(dparts[:, :N], exl)
        oparts = _sc_pass_b(idx2, dst_e, ex_t, rden_t,
                            xs.reshape(NC * N, FH))
        outl = _tc_outl(attl, xs)
        hc = _tc_post(oparts[:, :N], outl, p['bias'][None, :],
                      p['bn_g'][None, :], p['bn_b'][None, :])
        h = jnp.concatenate([h, hc], axis=1)

    return _tc_final(h, params['W1'], params['b1'][None, :],
                     params['bn_g'][None, :], params['bn_b'][None, :],
                     params['W2'], params['b2'][None, :],
                     batch[:, None].astype(jnp.int32))


# speedup vs baseline: 2.3733x; 1.0741x over previous
import functools

import jax
import jax.numpy as jnp
from jax import lax
from jax.experimental import pallas as pl
from jax.experimental.pallas import tpu as pltpu
from jax.experimental.pallas import tpu_sc as plsc

N = 10000
E = 320000
DIN = 128
DE = 4
H = 10
C = 64
HP = 16
EMB = 10
NL = 3
DOUT = 128
G = 64
F = H * C

NC = 2
NS = 16
NWK = NC * NS
E2 = 327680
EW = E2 // NWK
WIN = 128
NWIN = EW // WIN
NP = 10240
RPT = NP // NS
CH = C // NC
FH = H * CH
EW2 = E2 // NS
NWIN2 = EW2 // WIN

_f32 = jnp.float32

_XPERM = [h * C + p * (C // NC) + i
          for p in range(NC) for h in range(H) for i in range(C // NC)]


def _gelu(x):
    return 0.5 * x * (1.0 + lax.erf(x * (2.0 ** -0.5)))


def _embed_body(x_ref, w_ref, b_ref, o_ref):
    o_ref[...] = x_ref[...] @ w_ref[...] + b_ref[...]


def _tc_embed(x, w, b):
    return pl.pallas_call(
        _embed_body,
        out_shape=jax.ShapeDtypeStruct((N, EMB), _f32),
    )(x, w, b)


def _ea_body(eat_ref, me_ref, aeloop_ref, rn_ref):
    eat = eat_ref[...]
    aeloop_ref[...] = (jnp.sum(eat, axis=1, keepdims=True) / E).T @ me_ref[...]
    rn = jnp.sqrt(jnp.max(jnp.sum(eat * eat, axis=0)))
    rn_ref[...] = jnp.full((1, 1), 0.0, _f32) + rn


def _tc_ea(eat, me_all):
    return pl.pallas_call(
        _ea_body,
        out_shape=(
            jax.ShapeDtypeStruct((1, NL * HP), _f32),
            jax.ShapeDtypeStruct((1, 1), _f32),
        ),
    )(eat, me_all)


_BE = 8192


def _aedge_body(ea_ref, me_ref, o_ref):
    o = ea_ref[...] @ me_ref[...]
    rid = pl.program_id(0) * _BE + lax.broadcasted_iota(jnp.int32, (_BE, HP), 0)
    o_ref[...] = jnp.where(rid < E, o, -120.0)


def _tc_aedge(ea_p, me):
    return pl.pallas_call(
        _aedge_body,
        grid=(E2 // _BE,),
        in_specs=[
            pl.BlockSpec((_BE, DE), lambda i: (i, 0)),
            pl.BlockSpec((DE, HP), lambda i: (0, 0)),
        ],
        out_specs=pl.BlockSpec((_BE, HP), lambda i: (i, 0)),
        out_shape=jax.ShapeDtypeStruct((E2, HP), _f32),
    )(ea_p, me)


def _xs_body(h_ref, w_ref, o_ref):
    o_ref[...] = h_ref[...] @ w_ref[...]


def _tc_xs(h, w):
    return pl.pallas_call(
        _xs_body,
        out_shape=jax.ShapeDtypeStruct((N, F), _f32),
    )(h, w)


def _att_body(h_ref, wsd_ref, rn_ref, menorm_ref, aeloop_ref,
              asd_ref, m_ref, exl_ref):
    h = h_ref[...]
    asd = h @ wsd_ref[...]
    asd_ref[...] = asd
    asrc = asd[:, :HP]
    adst = asd[:, HP:]
    srcmax = jnp.max(asrc, axis=0, keepdims=True)
    dstmax = jnp.max(adst, axis=0, keepdims=True)
    bound = srcmax + dstmax + rn_ref[0, 0] * menorm_ref[...]
    m = jnp.maximum(bound, 0.2 * bound)
    m_ref[...] = m
    al = asrc + adst + aeloop_ref[...]
    al = jnp.maximum(al, 0.2 * al)
    exl = jnp.exp(al - m)
    lane = lax.broadcasted_iota(jnp.int32, (N, HP), 1)
    exl_ref[...] = jnp.where(lane < H, exl, 0.0)


def _tc_att(h, wsd, rn, menorm, aeloop):
    return pl.pallas_call(
        _att_body,
        out_shape=(
            jax.ShapeDtypeStruct((N, 2 * HP), _f32),
            jax.ShapeDtypeStruct((1, HP), _f32),
            jax.ShapeDtypeStruct((N, HP), _f32),
        ),
    )(h, wsd, rn, menorm, aeloop)


def _rden_body(dp_ref, exl_ref, rden_ref, attl_ref):
    exl = exl_ref[...]
    den = dp_ref[0] + dp_ref[1] + exl
    rden = 1.0 / (den + 1e-16)
    rden_ref[...] = rden
    attl_ref[...] = exl * rden


def _tc_rden(dparts, exl):
    return pl.pallas_call(
        _rden_body,
        out_shape=(
            jax.ShapeDtypeStruct((N, HP), _f32),
            jax.ShapeDtypeStruct((N, HP), _f32),
        ),
    )(dparts, exl)


def _outl_body(attl_ref, xs_ref, o_ref):
    attl = attl_ref[...]
    halves = []
    for p in range(NC):
        acc = attl[:, 0:1] * xs_ref[:, p * FH:p * FH + CH]
        for hh in range(1, H):
            acc = acc + attl[:, hh:hh + 1] * xs_ref[:, p * FH + hh * CH:p * FH + (hh + 1) * CH]
        halves.append(acc)
    o_ref[...] = jnp.concatenate(halves, axis=1)


_BN = 1000


def _tc_outl(attl, xs):
    return pl.pallas_call(
        _outl_body,
        grid=(N // _BN,),
        in_specs=[
            pl.BlockSpec((_BN, HP), lambda i: (i, 0)),
            pl.BlockSpec((_BN, F), lambda i: (i, 0)),
        ],
        out_specs=pl.BlockSpec((_BN, C), lambda i: (i, 0)),
        out_shape=jax.ShapeDtypeStruct((N, C), _f32),
    )(attl, xs)


def _post_body(op_ref, ol_ref, bias_ref, g_ref, b_ref, o_ref):
    tot = (jnp.concatenate([op_ref[0], op_ref[1]], axis=1)
           + ol_ref[...]) / H + bias_ref[...]
    mu = jnp.mean(tot, axis=0, keepdims=True)
    var = jnp.mean((tot - mu) * (tot - mu), axis=0, keepdims=True)
    xn = (tot - mu) / jnp.sqrt(var + 1e-5) * g_ref[...] + b_ref[...]
    o_ref[...] = _gelu(xn)


def _tc_post(oparts, outl, bias, bn_g, bn_b):
    return pl.pallas_call(
        _post_body,
        out_shape=jax.ShapeDtypeStruct((N, C), _f32),
    )(oparts, outl, bias, bn_g, bn_b)


def _final_body(h_ref, w1_ref, b1_ref, g_ref, b_ref, w2_ref, b2_ref,
                batch_ref, o_ref):
    y = h_ref[...] @ w1_ref[...] + b1_ref[...]
    mu = jnp.mean(y, axis=0, keepdims=True)
    var = jnp.mean((y - mu) * (y - mu), axis=0, keepdims=True)
    y = (y - mu) / jnp.sqrt(var + 1e-5) * g_ref[...] + b_ref[...]
    y = _gelu(y)
    y = y @ w2_ref[...] + b2_ref[...]
    gid = lax.broadcasted_iota(jnp.int32, (N, G), 1)
    p = jnp.where(batch_ref[...] == gid, 1.0, 0.0)
    s = lax.dot_general(p, y, (((0,), (0,)), ((), ())))
    cnt = lax.dot_general(p, jnp.ones((N, 1), _f32), (((0,), (0,)), ((), ())))
    o_ref[...] = s / jnp.maximum(cnt, 1.0)


def _tc_final(h, w1, b1, g, b, w2, b2, batch_col):
    return pl.pallas_call(
        _final_body,
        out_shape=jax.ShapeDtypeStruct((G, DOUT), _f32),
    )(h, w1, b1, g, b, w2, b2, batch_col)


_MESH = plsc.VectorSubcoreMesh(core_axis_name="c", subcore_axis_name="s")


@functools.partial(
    pl.kernel,
    out_type=(
        jax.ShapeDtypeStruct((E2, HP), _f32),
        jax.ShapeDtypeStruct((NC, NP, HP), _f32),
    ),
    mesh=_MESH,
    compiler_params=pltpu.CompilerParams(use_tc_tiling_on_sc=False),
    scratch_types=[
        pltpu.VMEM_SHARED((NP, HP), _f32),
        pltpu.VMEM((WIN,), jnp.int32),
        pltpu.VMEM((WIN,), jnp.int32),
        pltpu.VMEM((WIN,), jnp.int32),
        pltpu.VMEM((WIN,), jnp.int32),
        pltpu.VMEM((WIN, HP), _f32),
        pltpu.VMEM((WIN, HP), _f32),
        pltpu.VMEM((1, HP), _f32),
        pltpu.VMEM((WIN, HP), _f32),
        pltpu.VMEM((WIN, HP), _f32),
        pltpu.VMEM((WIN, HP), _f32),
        pltpu.VMEM((WIN, HP), _f32),
        pltpu.VMEM((WIN, HP), _f32),
        pltpu.VMEM((WIN, HP), _f32),
        pltpu.VMEM((WIN, HP), _f32),
        pltpu.SemaphoreType.DMA,
        pltpu.SemaphoreType.DMA,
        pltpu.SemaphoreType.DMA,
        pltpu.SemaphoreType.DMA,
        pltpu.SemaphoreType.DMA,
        pltpu.SemaphoreType.DMA,
        pltpu.SemaphoreType.DMA,
        pltpu.SemaphoreType.DMA,
        pltpu.SemaphoreType.DMA,
        pltpu.SemaphoreType.DMA,
        pltpu.SemaphoreType.DMA,
        pltpu.SemaphoreType.DMA,
        pltpu.SemaphoreType.DMA,
        pltpu.SemaphoreType.DMA,
    ],
)
def _sc_pass_a(src_e, dst_e, ae, m, asrc_t, adst_t,
               ex_out, dparts,
               den_sp, src_v0, src_v1, dst_v0, dst_v1, ae_v0, ae_v1, m_v,
               as_v0, as_v1, ad_v0, ad_v1, ex_v0, ex_v1, zero_c,
               slA0, slB0, slC0, slA1, slB1, slC1,
               sgA0, sgB0, sgA1, sgB1, swA0, swB0, swA1, swB1):
    cid = lax.axis_index("c")
    sid = lax.axis_index("s")
    wid = sid * NC + cid
    base = wid * EW

    src_v = (src_v0, src_v1)
    dst_v = (dst_v0, dst_v1)
    ae_v = (ae_v0, ae_v1)
    as_v = (as_v0, as_v1)
    ad_v = (ad_v0, ad_v1)
    ex_v = (ex_v0, ex_v1)
    slA = (slA0, slA1)
    slB = (slB0, slB1)
    slC = (slC0, slC1)
    sgA = (sgA0, sgA1)
    sgB = (sgB0, sgB1)
    swA = (swA0, swA1)
    swB = (swB0, swB1)

    zvec = jnp.zeros((16,), _f32)

    def zb(r, _):
        zero_c[r, :] = zvec
        return 0

    lax.fori_loop(0, WIN, zb, 0)
    for k in range(RPT // WIN):
        pltpu.sync_copy(zero_c, den_sp.at[pl.ds(sid * RPT + k * WIN, WIN)])

    pltpu.sync_copy(m, m_v)
    plsc.subcore_barrier()

    mrow = m_v[0]
    lane = lax.iota(jnp.int32, 16)
    lmask = lane < H

    def issue_loads(w, p):
        eb = base + w * WIN
        pltpu.async_copy(src_e.at[pl.ds(eb, WIN)], src_v[p], slA[p])
        pltpu.async_copy(dst_e.at[pl.ds(eb, WIN)], dst_v[p], slB[p])
        pltpu.async_copy(ae.at[pl.ds(eb, WIN)], ae_v[p], slC[p])

    def wait_loads(p):
        pltpu.make_async_copy(src_e.at[pl.ds(0, WIN)], src_v[p], slA[p]).wait()
        pltpu.make_async_copy(dst_e.at[pl.ds(0, WIN)], dst_v[p], slB[p]).wait()
        pltpu.make_async_copy(ae.at[pl.ds(0, WIN)], ae_v[p], slC[p]).wait()

    def issue_gathers(p):
        pltpu.async_copy(asrc_t.at[src_v[p]], as_v[p], sgA[p])
        pltpu.async_copy(adst_t.at[dst_v[p]], ad_v[p], sgB[p])

    def wait_gathers(p):
        pltpu.make_async_copy(asrc_t.at[src_v[p]], as_v[p], sgA[p]).wait()
        pltpu.make_async_copy(adst_t.at[dst_v[p]], ad_v[p], sgB[p]).wait()

    issue_loads(0, 0)
    issue_loads(1, 1)
    wait_loads(0)
    issue_gathers(0)

    def wbody(w2, _):
        for p in range(2):
            w = w2 * 2 + p
            q = 1 - p
            wait_gathers(p)

            @pl.when(w + 1 < NWIN)
            def _():
                wait_loads(q)
                issue_gathers(q)

            @pl.when(w >= 2)
            def _():
                eb0 = base
                pltpu.make_async_copy(
                    ex_v[p], ex_out.at[pl.ds(eb0, WIN)], swA[p]).wait()
                pltpu.make_async_copy(
                    ex_v[p], den_sp.at[dst_v[p]], swB[p]).wait()

            def ebody(j, _):
                a = as_v[p][j] + ad_v[p][j] + ae_v[p][j]
                a = jnp.maximum(a, 0.2 * a)
                e = jnp.exp(a - mrow)
                ex_v[p][j, :] = jnp.where(lmask, e, 0.0)
                return 0

            lax.fori_loop(0, WIN, ebody, 0)
            eb = base + w * WIN
            pltpu.async_copy(ex_v[p], ex_out.at[pl.ds(eb, WIN)], swA[p])
            pltpu.async_copy(ex_v[p], den_sp.at[dst_v[p]], swB[p], add=True)

            @pl.when(w + 2 < NWIN)
            def _():
                issue_loads(w + 2, p)
        return 0

    lax.fori_loop(0, NWIN // 2, wbody, 0)

    for p in range(2):
        pltpu.make_async_copy(ex_v[p], ex_out.at[pl.ds(base, WIN)],
                              swA[p]).wait()
        pltpu.make_async_copy(ex_v[p], den_sp.at[dst_v[p]], swB[p]).wait()

    plsc.subcore_barrier()
    pltpu.sync_copy(den_sp.at[pl.ds(sid * RPT, RPT)],
                    dparts.at[cid, pl.ds(sid * RPT, RPT)])


@functools.partial(
    pl.kernel,
    out_type=jax.ShapeDtypeStruct((NC, NP, CH), _f32),
    mesh=_MESH,
    compiler_params=pltpu.CompilerParams(use_tc_tiling_on_sc=False),
    scratch_types=[
        pltpu.VMEM_SHARED((NP, CH), _f32),
        pltpu.VMEM((WIN,), jnp.int32),
        pltpu.VMEM((WIN,), jnp.int32),
        pltpu.VMEM((WIN,), jnp.int32),
        pltpu.VMEM((WIN,), jnp.int32),
        pltpu.VMEM((WIN, HP), _f32),
        pltpu.VMEM((WIN, HP), _f32),
        pltpu.VMEM((WIN, HP), _f32),
        pltpu.VMEM((WIN, HP), _f32),
        pltpu.VMEM((WIN, FH), _f32),
        pltpu.VMEM((WIN, FH), _f32),
        pltpu.VMEM((WIN, CH), _f32),
        pltpu.VMEM((WIN, CH), _f32),
        pltpu.SemaphoreType.DMA,
        pltpu.SemaphoreType.DMA,
        pltpu.SemaphoreType.DMA,
        pltpu.SemaphoreType.DMA,
        pltpu.SemaphoreType.DMA,
        pltpu.SemaphoreType.DMA,
        pltpu.SemaphoreType.DMA,
        pltpu.SemaphoreType.DMA,
        pltpu.SemaphoreType.DMA,
        pltpu.SemaphoreType.DMA,
    ],
)
def _sc_pass_b(idx2, dst_e, ex_t, rden_t, xsr2,
               oparts,
               out_sp, dst_v0, dst_v1, idx_v0, idx_v1, ex_v0, ex_v1,
               rd_v0, rd_v1, xs_v0, xs_v1, os_v, zero_c,
               slA0, slB0, slC0, slA1, slB1, slC1, sgA0, sgB0, sgA1, sgB1):
    cid = lax.axis_index("c")
    sid = lax.axis_index("s")
    base = sid * EW2

    dst_v = (dst_v0, dst_v1)
    idx_v = (idx_v0, idx_v1)
    ex_v = (ex_v0, ex_v1)
    rd_v = (rd_v0, rd_v1)
    xs_v = (xs_v0, xs_v1)
    slA = (slA0, slA1)
    slB = (slB0, slB1)
    slC = (slC0, slC1)
    sgA = (sgA0, sgA1)
    sgB = (sgB0, sgB1)

    zvec = jnp.zeros((16,), _f32)

    def zb(r, _):
        for q in range(CH // 16):
            zero_c[r, pl.ds(q * 16, 16)] = zvec
        return 0

    lax.fori_loop(0, WIN, zb, 0)
    for k in range(RPT // WIN):
        pltpu.sync_copy(zero_c, out_sp.at[pl.ds(sid * RPT + k * WIN, WIN)])
    plsc.subcore_barrier()

    def issue_loads(w, p):
        eb = base + w * WIN
        pltpu.async_copy(dst_e.at[pl.ds(eb, WIN)], dst_v[p], slA[p])
        pltpu.async_copy(idx2.at[cid, pl.ds(eb, WIN)], idx_v[p], slB[p])
        pltpu.async_copy(ex_t.at[pl.ds(eb, WIN)], ex_v[p], slC[p])

    def wait_loads(p):
        pltpu.make_async_copy(dst_e.at[pl.ds(0, WIN)], dst_v[p], slA[p]).wait()
        pltpu.make_async_copy(idx2.at[0, pl.ds(0, WIN)], idx_v[p], slB[p]).wait()
        pltpu.make_async_copy(ex_t.at[pl.ds(0, WIN)], ex_v[p], slC[p]).wait()

    def issue_gathers(p):
        pltpu.async_copy(rden_t.at[dst_v[p]], rd_v[p], sgA[p])
        pltpu.async_copy(xsr2.at[idx_v[p]], xs_v[p], sgB[p])

    def wait_gathers(p):
        pltpu.make_async_copy(rden_t.at[dst_v[p]], rd_v[p], sgA[p]).wait()
        pltpu.make_async_copy(xsr2.at[idx_v[p]], xs_v[p], sgB[p]).wait()

    issue_loads(0, 0)
    issue_loads(1, 1)
    wait_loads(0)
    issue_gathers(0)

    def wbody(w2, _):
        for p in range(2):
            w = w2 * 2 + p
            q = 1 - p
            wait_gathers(p)

            @pl.when(w + 1 < NWIN2)
            def _():
                wait_loads(q)
                issue_gathers(q)

            def ebody(j, _):
                att = ex_v[p][j] * rd_v[p][j]
                acc0 = jnp.zeros((16,), _f32)
                acc1 = jnp.zeros((16,), _f32)
                for hh in range(H):
                    ah = att[hh]
                    o = hh * CH
                    acc0 = acc0 + ah * xs_v[p][j, pl.ds(o, 16)]
                    acc1 = acc1 + ah * xs_v[p][j, pl.ds(o + 16, 16)]
                os_v[j, pl.ds(0, 16)] = acc0
                os_v[j, pl.ds(16, 16)] = acc1
                return 0

            lax.fori_loop(0, WIN, ebody, 0)
            pltpu.sync_copy(os_v, out_sp.at[dst_v[p]], add=True)

            @pl.when(w + 2 < NWIN2)
            def _():
                issue_loads(w + 2, p)
        return 0

    lax.fori_loop(0, NWIN2 // 2, wbody, 0)

    plsc.subcore_barrier()
    pltpu.sync_copy(out_sp.at[pl.ds(sid * RPT, RPT)],
                    oparts.at[cid, pl.ds(sid * RPT, RPT)])


def _pad_cols(a, width):
    return jnp.concatenate(
        [a, jnp.zeros((a.shape[0], width - a.shape[1]), a.dtype)], axis=1)


def kernel(x, edge_index, edge_attr, batch, params):
    me_l = []
    menorm_l = []
    wsd_l = []
    for l in range(NL):
        p = params['convs'][l]
        w_e = p['W_edge'].reshape(DE, H, C)
        me = jnp.einsum('dhc,hc->dh', w_e, p['att_edge'])
        me_l.append(_pad_cols(me, HP))
        menorm_l.append(_pad_cols(
            jnp.sqrt(jnp.sum(me * me, axis=0))[None, :], HP))
        w3 = p['W'].reshape(-1, H, C)
        ws = jnp.einsum('dhc,hc->dh', w3, p['att_src'])
        wd = jnp.einsum('dhc,hc->dh', w3, p['att_dst'])
        wsd_l.append(jnp.concatenate(
            [_pad_cols(ws, HP), _pad_cols(wd, HP)], axis=1))

    me_all = jnp.concatenate(me_l, axis=1)
    eat = edge_attr.T
    aeloop_all, rn = _tc_ea(eat, me_all)

    pad = E2 - E
    spread = jnp.arange(pad, dtype=jnp.int32) * 13 % N
    src_e = jnp.concatenate([edge_index[0], spread])
    dst_e = jnp.concatenate([edge_index[1], spread])
    idx2 = jnp.stack([src_e * 2, src_e * 2 + 1])
    ea_p = jnp.concatenate([edge_attr, jnp.zeros((pad, DE), _f32)], axis=0)
    h = _tc_embed(x, params['W_emb'], params['b_emb'][None, :])

    for l in range(NL):
        p = params['convs'][l]
        aeloop = aeloop_all[:, l * HP:(l + 1) * HP]
        xs = _tc_xs(h, p['W'][:, _XPERM])
        ae = _tc_aedge(ea_p, me_l[l])
        asd, m, exl = _tc_att(h, wsd_l[l], rn, menorm_l[l], aeloop)
        asrc_t = asd[:, :HP]
        adst_t = asd[:, HP:]
        ex_t, dparts = _sc_pass_a(src_e, dst_e, ae, m, asrc_t, adst_t)
        rden_t, attl = _tc_rden(dparts[:, :N], exl)
        oparts = _sc_pass_b(idx2, dst_e, ex_t, rden_t,
                            xs.reshape(NC * N, FH))
        outl = _tc_outl(attl, xs)
        hc = _tc_post(oparts[:, :N], outl, p['bias'][None, :],
                      p['bn_g'][None, :], p['bn_b'][None, :])
        h = jnp.concatenate([h, hc], axis=1)

    return _tc_final(h, params['W1'], params['b1'][None, :],
                     params['bn_g'][None, :], params['bn_b'][None, :],
                     params['W2'], params['b2'][None, :],
                     batch[:, None].astype(jnp.int32))
